# Initial kernel scaffold; baseline (speedup 1.0000x reference)
import jax
import jax.numpy as jnp
from jax.experimental import pallas as pl


def kernel(xyz, features, W1, b1, g1, be1, W2, b2, g2, be2):
    raise NotImplementedError("write your pallas kernel here")


# speedup vs baseline: 588.1333x; 588.1333x over previous
import functools

import jax
import jax.numpy as jnp
from jax import lax
from jax.experimental import pallas as pl
from jax.experimental.pallas import tpu as pltpu
from jax.experimental.pallas import tpu_sc as plsc

B = 8
N = 4096
S = 1024
K = 16
DIN = 128
DOUT = 256
M = B * S * K


def _fps_body(x_ref, y_ref, z_ref, cx_ref, cy_ref, cz_ref):
    x = x_ref[...]
    y = y_ref[...]
    z = z_ref[...]
    iota_n = lax.broadcasted_iota(jnp.int32, (B, N), 1)
    iota_s = lax.broadcasted_iota(jnp.int32, (B, S), 1)

    def body(i, carry):
        dist, far, cx_acc, cy_acc, cz_acc = carry
        m = iota_n == far
        cx = jnp.sum(jnp.where(m, x, 0.0), axis=1, keepdims=True)
        cy = jnp.sum(jnp.where(m, y, 0.0), axis=1, keepdims=True)
        cz = jnp.sum(jnp.where(m, z, 0.0), axis=1, keepdims=True)
        sel = iota_s == i
        cx_acc = jnp.where(sel, cx, cx_acc)
        cy_acc = jnp.where(sel, cy, cy_acc)
        cz_acc = jnp.where(sel, cz, cz_acc)
        d = (x - cx) ** 2 + (y - cy) ** 2 + (z - cz) ** 2
        dist = jnp.minimum(dist, d)
        mx = jnp.max(dist, axis=1, keepdims=True)
        far = jnp.min(jnp.where(dist == mx, iota_n, N), axis=1, keepdims=True)
        return dist, far, cx_acc, cy_acc, cz_acc

    init = (
        jnp.full((B, N), 1e10, jnp.float32),
        jnp.zeros((B, 1), jnp.int32),
        jnp.zeros((B, S), jnp.float32),
        jnp.zeros((B, S), jnp.float32),
        jnp.zeros((B, S), jnp.float32),
    )
    _, _, cx_acc, cy_acc, cz_acc = lax.fori_loop(0, S, body, init)
    cx_ref[...] = cx_acc
    cy_ref[...] = cy_acc
    cz_ref[...] = cz_acc


def _fps(x, y, z):
    return pl.pallas_call(
        _fps_body,
        out_shape=[jax.ShapeDtypeStruct((B, S), jnp.float32)] * 3,
    )(x, y, z)


SBLK = 256


def _knn_body(cx_ref, cy_ref, cz_ref, x_ref, y_ref, z_ref, idx_ref):
    cx = cx_ref[0]
    cy = cy_ref[0]
    cz = cz_ref[0]
    x = x_ref[0]
    y = y_ref[0]
    z = z_ref[0]
    d = (cx - x) ** 2 + (cy - y) ** 2 + (cz - z) ** 2
    iota_n = lax.broadcasted_iota(jnp.int32, (SBLK, N), 1)
    iota_k = lax.broadcasted_iota(jnp.int32, (SBLK, K), 1)
    idx_acc = jnp.zeros((SBLK, K), jnp.int32)
    for k in range(K):
        mn = jnp.min(d, axis=1, keepdims=True)
        amn = jnp.min(jnp.where(d == mn, iota_n, N), axis=1, keepdims=True)
        idx_acc = jnp.where(iota_k == k, amn, idx_acc)
        if k < K - 1:
            d = jnp.where(iota_n == amn, jnp.inf, d)
    idx_ref[0] = idx_acc


def _knn(cx, cy, cz, x, y, z):
    return pl.pallas_call(
        _knn_body,
        grid=(B, S // SBLK),
        in_specs=[
            pl.BlockSpec((1, SBLK, 1), lambda b, s: (b, s, 0)),
            pl.BlockSpec((1, SBLK, 1), lambda b, s: (b, s, 0)),
            pl.BlockSpec((1, SBLK, 1), lambda b, s: (b, s, 0)),
            pl.BlockSpec((1, 1, N), lambda b, s: (b, 0, 0)),
            pl.BlockSpec((1, 1, N), lambda b, s: (b, 0, 0)),
            pl.BlockSpec((1, 1, N), lambda b, s: (b, 0, 0)),
        ],
        out_specs=pl.BlockSpec((1, SBLK, K), lambda b, s: (b, s, 0)),
        out_shape=jax.ShapeDtypeStruct((B, S, K), jnp.int32),
    )(cx, cy, cz, x, y, z)


NW = 32
ROWS_PER_W = M // NW
CH = 128
NCHUNK = ROWS_PER_W // CH


def _sc_gather(table, gidx3):
    mesh = plsc.VectorSubcoreMesh(core_axis_name="c", subcore_axis_name="s")

    @functools.partial(
        pl.kernel,
        mesh=mesh,
        out_type=jax.ShapeDtypeStruct((M, DIN), jnp.float32),
        scratch_types=[
            pltpu.VMEM((NCHUNK, CH), jnp.int32),
            pltpu.VMEM((CH, DIN), jnp.float32),
            pltpu.SemaphoreType.DMA,
        ],
    )
    def gather_k(table_hbm, idx_hbm, out_hbm, idx_v, rows_v, sem):
        wid = lax.axis_index("s") * 2 + lax.axis_index("c")
        base = wid * ROWS_PER_W
        pltpu.sync_copy(idx_hbm.at[wid], idx_v)

        def body(j, _):
            pltpu.async_copy(table_hbm.at[idx_v.at[j]], rows_v, sem).wait()
            pltpu.sync_copy(rows_v, out_hbm.at[pl.ds(base + j * CH, CH)])
            return 0

        lax.fori_loop(0, NCHUNK, body, 0)

    return gather_k(table, gidx3)


RB_MOM = 2048


def _mom_body(g_ref, c_ref, s_ref):
    @pl.when(pl.program_id(0) == 0)
    def _():
        c_ref[...] = jnp.zeros_like(c_ref)
        s_ref[...] = jnp.zeros_like(s_ref)

    g = g_ref[...]
    c_ref[...] += lax.dot_general(
        g, g, (((0,), (0,)), ((), ())), preferred_element_type=jnp.float32
    )
    s_ref[...] += jnp.sum(g, axis=0, keepdims=True)


def _moments(g):
    return pl.pallas_call(
        _mom_body,
        grid=(M // RB_MOM,),
        in_specs=[pl.BlockSpec((RB_MOM, DIN), lambda i: (i, 0))],
        out_specs=[
            pl.BlockSpec((DIN, DIN), lambda i: (0, 0)),
            pl.BlockSpec((1, DIN), lambda i: (0, 0)),
        ],
        out_shape=[
            jax.ShapeDtypeStruct((DIN, DIN), jnp.float32),
            jax.ShapeDtypeStruct((1, DIN), jnp.float32),
        ],
    )(g)


def _fold_body(c_ref, s_ref, w1_ref, g1_ref, be1_ref, w1p_ref, b1p_ref):
    inv_m = jnp.float32(1.0 / M)
    mean = s_ref[...] * inv_m
    outer = lax.dot_general(
        mean, mean, (((0,), (0,)), ((), ())), preferred_element_type=jnp.float32
    )
    cc = c_ref[...] * inv_m - outer
    w1 = w1_ref[...]
    t = lax.dot_general(
        w1, cc, (((1,), (0,)), ((), ())), preferred_element_type=jnp.float32
    )
    var = jnp.sum(t * w1, axis=1, keepdims=True)
    mu_c = lax.dot_general(
        w1, mean, (((1,), (1,)), ((), ())), preferred_element_type=jnp.float32
    )
    scale = g1_ref[...] / jnp.sqrt(var + 1e-5)
    w1p_ref[...] = w1 * scale
    b1p_ref[...] = be1_ref[...] - mu_c * scale


def _fold_bn1(c, s, w1, g1, be1):
    return pl.pallas_call(
        _fold_body,
        out_shape=[
            jax.ShapeDtypeStruct((DOUT, DIN), jnp.float32),
            jax.ShapeDtypeStruct((DOUT, 1), jnp.float32),
        ],
    )(c, s, w1, g1, be1)


RB_MLP = 512


def _mlp_body(g_ref, w1p_ref, b1p_ref, w2_ref, b2_ref, y2_ref, s2_ref, q2_ref):
    @pl.when(pl.program_id(0) == 0)
    def _():
        s2_ref[...] = jnp.zeros_like(s2_ref)
        q2_ref[...] = jnp.zeros_like(q2_ref)

    g = g_ref[...]
    z1 = lax.dot_general(
        g, w1p_ref[...], (((1,), (1,)), ((), ())), preferred_element_type=jnp.float32
    )
    z1 = jnp.maximum(z1 + b1p_ref[...], 0.0)
    y2 = lax.dot_general(
        z1, w2_ref[...], (((1,), (1,)), ((), ())), preferred_element_type=jnp.float32
    )
    y2 = y2 + b2_ref[...]
    y2_ref[...] = y2
    s2_ref[...] += jnp.sum(y2, axis=0, keepdims=True)
    q2_ref[...] += jnp.sum(y2 * y2, axis=0, keepdims=True)


def _mlp(g, w1p, b1p, w2, b2):
    return pl.pallas_call(
        _mlp_body,
        grid=(M // RB_MLP,),
        in_specs=[
            pl.BlockSpec((RB_MLP, DIN), lambda i: (i, 0)),
            pl.BlockSpec((DOUT, DIN), lambda i: (0, 0)),
            pl.BlockSpec((1, DOUT), lambda i: (0, 0)),
            pl.BlockSpec((DOUT, DOUT), lambda i: (0, 0)),
            pl.BlockSpec((1, DOUT), lambda i: (0, 0)),
        ],
        out_specs=[
            pl.BlockSpec((RB_MLP, DOUT), lambda i: (i, 0)),
            pl.BlockSpec((1, DOUT), lambda i: (0, 0)),
            pl.BlockSpec((1, DOUT), lambda i: (0, 0)),
        ],
        out_shape=[
            jax.ShapeDtypeStruct((M, DOUT), jnp.float32),
            jax.ShapeDtypeStruct((1, DOUT), jnp.float32),
            jax.ShapeDtypeStruct((1, DOUT), jnp.float32),
        ],
    )(g, w1p, b1p, w2, b2)


RB_FIN = 2048


def _fin_body(y2_ref, s2_ref, q2_ref, g2_ref, be2_ref, out_ref):
    inv_m = jnp.float32(1.0 / M)
    m2 = s2_ref[...] * inv_m
    var2 = q2_ref[...] * inv_m - m2 * m2
    scale = g2_ref[...] / jnp.sqrt(var2 + 1e-5)
    shift = be2_ref[...] - m2 * scale
    z = jnp.maximum(y2_ref[...] * scale + shift, 0.0)
    z = z.reshape(RB_FIN // K, K, DOUT)
    out_ref[...] = jnp.max(z, axis=1)


def _finalize(y2, s2, q2, g2, be2):
    return pl.pallas_call(
        _fin_body,
        grid=(M // RB_FIN,),
        in_specs=[
            pl.BlockSpec((RB_FIN, DOUT), lambda i: (i, 0)),
            pl.BlockSpec((1, DOUT), lambda i: (0, 0)),
            pl.BlockSpec((1, DOUT), lambda i: (0, 0)),
            pl.BlockSpec((1, DOUT), lambda i: (0, 0)),
            pl.BlockSpec((1, DOUT), lambda i: (0, 0)),
        ],
        out_specs=pl.BlockSpec((RB_FIN // K, DOUT), lambda i: (i, 0)),
        out_shape=jax.ShapeDtypeStruct((M // K, DOUT), jnp.float32),
    )(y2, s2, q2, g2, be2)


def kernel(xyz, features, W1, b1, g1, be1, W2, b2, g2, be2):
    x = xyz[..., 0]
    y = xyz[..., 1]
    z = xyz[..., 2]
    cx, cy, cz = _fps(x, y, z)
    new_xyz = jnp.stack([cx, cy, cz], axis=-1)

    idx = _knn(
        cx[..., None], cy[..., None], cz[..., None],
        x[:, None, :], y[:, None, :], z[:, None, :],
    )

    gidx = idx + jnp.arange(B, dtype=jnp.int32)[:, None, None] * N
    gidx3 = gidx.reshape(NW, NCHUNK, CH)
    g = _sc_gather(features.reshape(B * N, DIN), gidx3)

    c_mat, s_vec = _moments(g)
    w1p, b1p = _fold_bn1(c_mat, s_vec, W1, g1[:, None], be1[:, None])
    y2, s2, q2 = _mlp(g, w1p, b1p.reshape(1, DOUT), W2, b2[None, :])
    feats = _finalize(y2, s2, q2, g2[None, :], be2[None, :])
    new_feats = feats.reshape(B, S, DOUT)
    return (new_xyz, new_feats)


# speedup vs baseline: 628.9875x; 1.0695x over previous
import functools

import jax
import jax.numpy as jnp
from jax import lax
from jax.experimental import pallas as pl
from jax.experimental.pallas import tpu as pltpu
from jax.experimental.pallas import tpu_sc as plsc

B = 8
N = 4096
S = 1024
K = 16
DIN = 128
DOUT = 256
M = B * S * K


def _tree_red(d, op, final):
    w = d.shape[1]
    while w > 128:
        w //= 2
        d = op(d[:, :w], d[:, w : 2 * w])
    return final(d, axis=1, keepdims=True)


def _rowmax(d):
    return _tree_red(d, jnp.maximum, jnp.max)


def _rowmin(d):
    return _tree_red(d, jnp.minimum, jnp.min)


def _rowsum(d):
    return _tree_red(d, jnp.add, jnp.sum)


def _fps_body(x_ref, y_ref, z_ref, cx_ref, cy_ref, cz_ref):
    x = x_ref[...]
    y = y_ref[...]
    z = z_ref[...]
    iota_n = lax.broadcasted_iota(jnp.int32, (B, N), 1)
    iota_s = lax.broadcasted_iota(jnp.int32, (B, S), 1)

    def body(i, carry):
        dist, far, cx_acc, cy_acc, cz_acc = carry
        m = iota_n == far
        cx = _rowsum(jnp.where(m, x, 0.0))
        cy = _rowsum(jnp.where(m, y, 0.0))
        cz = _rowsum(jnp.where(m, z, 0.0))
        sel = iota_s == i
        cx_acc = jnp.where(sel, cx, cx_acc)
        cy_acc = jnp.where(sel, cy, cy_acc)
        cz_acc = jnp.where(sel, cz, cz_acc)
        d = (x - cx) ** 2 + (y - cy) ** 2 + (z - cz) ** 2
        dist = jnp.minimum(dist, d)
        mx = _rowmax(dist)
        far = _rowmin(jnp.where(dist == mx, iota_n, N))
        return dist, far, cx_acc, cy_acc, cz_acc

    init = (
        jnp.full((B, N), 1e10, jnp.float32),
        jnp.zeros((B, 1), jnp.int32),
        jnp.zeros((B, S), jnp.float32),
        jnp.zeros((B, S), jnp.float32),
        jnp.zeros((B, S), jnp.float32),
    )
    _, _, cx_acc, cy_acc, cz_acc = lax.fori_loop(0, S, body, init)
    cx_ref[...] = cx_acc
    cy_ref[...] = cy_acc
    cz_ref[...] = cz_acc


def _fps(x, y, z):
    return pl.pallas_call(
        _fps_body,
        out_shape=[jax.ShapeDtypeStruct((B, S), jnp.float32)] * 3,
    )(x, y, z)


SBLK = 256


def _knn_body(cx_ref, cy_ref, cz_ref, x_ref, y_ref, z_ref, idx_ref):
    cx = cx_ref[0]
    cy = cy_ref[0]
    cz = cz_ref[0]
    x = x_ref[0]
    y = y_ref[0]
    z = z_ref[0]
    d = (cx - x) ** 2 + (cy - y) ** 2 + (cz - z) ** 2
    iota_n = lax.broadcasted_iota(jnp.int32, (SBLK, N), 1)
    iota_k = lax.broadcasted_iota(jnp.int32, (SBLK, K), 1)
    idx_acc = jnp.zeros((SBLK, K), jnp.int32)
    for k in range(K):
        mn = jnp.min(d, axis=1, keepdims=True)
        amn = jnp.min(jnp.where(d == mn, iota_n, N), axis=1, keepdims=True)
        idx_acc = jnp.where(iota_k == k, amn, idx_acc)
        if k < K - 1:
            d = jnp.where(iota_n == amn, jnp.inf, d)
    idx_ref[0] = idx_acc + pl.program_id(0) * N


def _knn(cx, cy, cz, x, y, z):
    return pl.pallas_call(
        _knn_body,
        grid=(B, S // SBLK),
        in_specs=[
            pl.BlockSpec((1, SBLK, 1), lambda b, s: (b, s, 0)),
            pl.BlockSpec((1, SBLK, 1), lambda b, s: (b, s, 0)),
            pl.BlockSpec((1, SBLK, 1), lambda b, s: (b, s, 0)),
            pl.BlockSpec((1, 1, N), lambda b, s: (b, 0, 0)),
            pl.BlockSpec((1, 1, N), lambda b, s: (b, 0, 0)),
            pl.BlockSpec((1, 1, N), lambda b, s: (b, 0, 0)),
        ],
        out_specs=pl.BlockSpec((1, SBLK, K), lambda b, s: (b, s, 0)),
        out_shape=jax.ShapeDtypeStruct((B, S, K), jnp.int32),
    )(cx, cy, cz, x, y, z)


NW = 32
ROWS_PER_W = M // NW
CH = 128
NCHUNK = ROWS_PER_W // CH


def _sc_gather(table, gidx3):
    mesh = plsc.VectorSubcoreMesh(core_axis_name="c", subcore_axis_name="s")

    @functools.partial(
        pl.kernel,
        mesh=mesh,
        out_type=jax.ShapeDtypeStruct((M, DIN), jnp.float32),
        scratch_types=[
            pltpu.VMEM((NCHUNK, CH), jnp.int32),
            pltpu.VMEM((CH, DIN), jnp.float32),
            pltpu.SemaphoreType.DMA,
        ],
    )
    def gather_k(table_hbm, idx_hbm, out_hbm, idx_v, rows_v, sem):
        wid = lax.axis_index("s") * 2 + lax.axis_index("c")
        base = wid * ROWS_PER_W
        pltpu.sync_copy(idx_hbm.at[wid], idx_v)

        def body(j, _):
            pltpu.async_copy(table_hbm.at[idx_v.at[j]], rows_v, sem).wait()
            pltpu.sync_copy(rows_v, out_hbm.at[pl.ds(base + j * CH, CH)])
            return 0

        lax.fori_loop(0, NCHUNK, body, 0)

    return gather_k(table, gidx3)


RB_MOM = 2048


def _mom_body(g_ref, c_ref, s_ref):
    @pl.when(pl.program_id(0) == 0)
    def _():
        c_ref[...] = jnp.zeros_like(c_ref)
        s_ref[...] = jnp.zeros_like(s_ref)

    g = g_ref[...]
    c_ref[...] += lax.dot_general(
        g, g, (((0,), (0,)), ((), ())), preferred_element_type=jnp.float32
    )
    s_ref[...] += jnp.sum(g, axis=0, keepdims=True)


def _moments(g):
    return pl.pallas_call(
        _mom_body,
        grid=(M // RB_MOM,),
        in_specs=[pl.BlockSpec((RB_MOM, DIN), lambda i: (i, 0))],
        out_specs=[
            pl.BlockSpec((DIN, DIN), lambda i: (0, 0)),
            pl.BlockSpec((1, DIN), lambda i: (0, 0)),
        ],
        out_shape=[
            jax.ShapeDtypeStruct((DIN, DIN), jnp.float32),
            jax.ShapeDtypeStruct((1, DIN), jnp.float32),
        ],
    )(g)


RB_MLP = 1024


def _mlp_body(
    g_ref, c_ref, s_ref, w1_ref, g1_ref, be1_ref, w2_ref, b2_ref,
    y2_ref, s2_ref, q2_ref, w1p_ref, b1p_ref,
):
    @pl.when(pl.program_id(0) == 0)
    def _():
        s2_ref[...] = jnp.zeros_like(s2_ref)
        q2_ref[...] = jnp.zeros_like(q2_ref)
        inv_m = jnp.float32(1.0 / M)
        mean = s_ref[...] * inv_m
        outer = lax.dot_general(
            mean, mean, (((0,), (0,)), ((), ())),
            preferred_element_type=jnp.float32,
        )
        cc = c_ref[...] * inv_m - outer
        w1 = w1_ref[...]
        t = lax.dot_general(
            w1, cc, (((1,), (0,)), ((), ())), preferred_element_type=jnp.float32
        )
        var = jnp.sum(t * w1, axis=1, keepdims=True)
        mu_c = lax.dot_general(
            w1, mean, (((1,), (1,)), ((), ())), preferred_element_type=jnp.float32
        )
        scale = g1_ref[...] / jnp.sqrt(var + 1e-5)
        w1p_ref[...] = w1 * scale
        b1p_ref[...] = jnp.transpose(be1_ref[...] - mu_c * scale)

    g = g_ref[...]
    z1 = lax.dot_general(
        g, w1p_ref[...], (((1,), (1,)), ((), ())), preferred_element_type=jnp.float32
    )
    z1 = jnp.maximum(z1 + b1p_ref[...], 0.0)
    y2 = lax.dot_general(
        z1, w2_ref[...], (((1,), (1,)), ((), ())), preferred_element_type=jnp.float32
    )
    y2 = y2 + b2_ref[...]
    y2_ref[...] = y2
    s2_ref[...] += jnp.sum(y2, axis=0, keepdims=True)
    q2_ref[...] += jnp.sum(y2 * y2, axis=0, keepdims=True)


def _mlp(g, c_mat, s_vec, w1, g1, be1, w2, b2):
    return pl.pallas_call(
        _mlp_body,
        grid=(M // RB_MLP,),
        in_specs=[
            pl.BlockSpec((RB_MLP, DIN), lambda i: (i, 0)),
            pl.BlockSpec((DIN, DIN), lambda i: (0, 0)),
            pl.BlockSpec((1, DIN), lambda i: (0, 0)),
            pl.BlockSpec((DOUT, DIN), lambda i: (0, 0)),
            pl.BlockSpec((DOUT, 1), lambda i: (0, 0)),
            pl.BlockSpec((DOUT, 1), lambda i: (0, 0)),
            pl.BlockSpec((DOUT, DOUT), lambda i: (0, 0)),
            pl.BlockSpec((1, DOUT), lambda i: (0, 0)),
        ],
        out_specs=[
            pl.BlockSpec((RB_MLP, DOUT), lambda i: (i, 0)),
            pl.BlockSpec((1, DOUT), lambda i: (0, 0)),
            pl.BlockSpec((1, DOUT), lambda i: (0, 0)),
        ],
        out_shape=[
            jax.ShapeDtypeStruct((M, DOUT), jnp.float32),
            jax.ShapeDtypeStruct((1, DOUT), jnp.float32),
            jax.ShapeDtypeStruct((1, DOUT), jnp.float32),
        ],
        scratch_shapes=[
            pltpu.VMEM((DOUT, DIN), jnp.float32),
            pltpu.VMEM((1, DOUT), jnp.float32),
        ],
    )(g, c_mat, s_vec, w1, g1, be1, w2, b2)


RB_FIN = 2048


def _fin_body(y2_ref, s2_ref, q2_ref, g2_ref, be2_ref, out_ref):
    inv_m = jnp.float32(1.0 / M)
    m2 = s2_ref[...] * inv_m
    var2 = q2_ref[...] * inv_m - m2 * m2
    scale = g2_ref[...] / jnp.sqrt(var2 + 1e-5)
    shift = be2_ref[...] - m2 * scale
    z = jnp.maximum(y2_ref[...] * scale + shift, 0.0)
    z = z.reshape(RB_FIN // K, K, DOUT)
    out_ref[...] = jnp.max(z, axis=1)


def _finalize(y2, s2, q2, g2, be2):
    return pl.pallas_call(
        _fin_body,
        grid=(M // RB_FIN,),
        in_specs=[
            pl.BlockSpec((RB_FIN, DOUT), lambda i: (i, 0)),
            pl.BlockSpec((1, DOUT), lambda i: (0, 0)),
            pl.BlockSpec((1, DOUT), lambda i: (0, 0)),
            pl.BlockSpec((1, DOUT), lambda i: (0, 0)),
            pl.BlockSpec((1, DOUT), lambda i: (0, 0)),
        ],
        out_specs=pl.BlockSpec((RB_FIN // K, DOUT), lambda i: (i, 0)),
        out_shape=jax.ShapeDtypeStruct((M // K, DOUT), jnp.float32),
    )(y2, s2, q2, g2, be2)


def kernel(xyz, features, W1, b1, g1, be1, W2, b2, g2, be2):
    x = xyz[..., 0]
    y = xyz[..., 1]
    z = xyz[..., 2]
    cx, cy, cz = _fps(x, y, z)
    new_xyz = jnp.stack([cx, cy, cz], axis=-1)

    gidx = _knn(
        cx[..., None], cy[..., None], cz[..., None],
        x[:, None, :], y[:, None, :], z[:, None, :],
    )
    g = _sc_gather(features.reshape(B * N, DIN), gidx.reshape(NW, NCHUNK, CH))

    c_mat, s_vec = _moments(g)
    y2, s2, q2 = _mlp(g, c_mat, s_vec, W1, g1[:, None], be1[:, None], W2, b2[None, :])
    feats = _finalize(y2, s2, q2, g2[None, :], be2[None, :])
    new_feats = feats.reshape(B, S, DOUT)
    return (new_xyz, new_feats)
